# baseline (device time: 19113 ns/iter reference)
import jax
import jax.numpy as jnp
from jax import lax
from jax.experimental import pallas as pl
from jax.experimental.pallas import tpu as pltpu

N_CHUNKS = 8


def kernel(x):
    m, n = x.shape
    q = m // N_CHUNKS
    half = N_CHUNKS // 2

    def body(x_hbm, out_hbm, xv, acc, recv_buf, send_sems, recv_sems,
             in_sem, out_sems):
        my_x = lax.axis_index("x")
        my_y = lax.axis_index("y")
        x_nbr = (1 - my_x, my_y)
        y_nbr = (my_x, 1 - my_y)

        in_copy = pltpu.make_async_copy(x_hbm, xv, in_sem)
        in_copy.start()

        barrier_sem = pltpu.get_barrier_semaphore()
        for nbr in (x_nbr, y_nbr):
            pl.semaphore_signal(
                barrier_sem, inc=1,
                device_id=nbr, device_id_type=pl.DeviceIdType.MESH,
            )
        pl.semaphore_wait(barrier_sem, 2)
        in_copy.wait()

        def chunk(ref, c):
            return ref.at[pl.ds(c * q, q), :]

        def mk_rdma(phase, c, nbr):
            src = chunk(xv if phase == 0 else acc, c)
            sem = N_CHUNKS * phase + c
            return pltpu.make_async_remote_copy(
                src_ref=src,
                dst_ref=recv_buf.at[phase, c],
                send_sem=send_sems.at[sem],
                recv_sem=recv_sems.at[sem],
                device_id=nbr,
                device_id_type=pl.DeviceIdType.MESH,
            )

        nbr0 = [x_nbr] * half + [y_nbr] * half
        nbr1 = [y_nbr] * half + [x_nbr] * half
        order = [c for pair in zip(range(half), range(half, N_CHUNKS))
                 for c in pair]

        p0 = {}
        for c in order:
            p0[c] = mk_rdma(0, c, nbr0[c])
            p0[c].start()
        p1 = {}
        for c in order:
            p0[c].wait()
            chunk(acc, c)[...] = chunk(xv, c)[...] + recv_buf[0, c]
            p1[c] = mk_rdma(1, c, nbr1[c])
            p1[c].start()
        stores = {}
        for c in order:
            p1[c].wait()
            chunk(acc, c)[...] += recv_buf[1, c]
            stores[c] = pltpu.make_async_copy(
                chunk(acc, c), chunk(out_hbm, c), out_sems.at[c]
            )
            stores[c].start()
        for c in order:
            stores[c].wait()

    return pl.pallas_call(
        body,
        out_shape=jax.ShapeDtypeStruct((m, n), jnp.float32),
        in_specs=[pl.BlockSpec(memory_space=pl.ANY)],
        out_specs=pl.BlockSpec(memory_space=pl.ANY),
        scratch_shapes=[
            pltpu.VMEM((m, n), jnp.float32),
            pltpu.VMEM((m, n), jnp.float32),
            pltpu.VMEM((2, N_CHUNKS, q, n), jnp.float32),
            pltpu.SemaphoreType.DMA((2 * N_CHUNKS,)),
            pltpu.SemaphoreType.DMA((2 * N_CHUNKS,)),
            pltpu.SemaphoreType.DMA,
            pltpu.SemaphoreType.DMA((N_CHUNKS,)),
        ],
        compiler_params=pltpu.CompilerParams(collective_id=0),
    )(x)


# device time: 19106 ns/iter; 1.0004x vs baseline; 1.0004x over previous
import jax
import jax.numpy as jnp
from jax import lax
from jax.experimental import pallas as pl
from jax.experimental.pallas import tpu as pltpu

N_CHUNKS = 8


def kernel(x):
    m, n = x.shape
    q = m // N_CHUNKS
    half = N_CHUNKS // 2

    def body(x_hbm, out_hbm, xv, acc, recv_buf, send_sems, recv_sems,
             in_sem, out_sems):
        my_x = lax.axis_index("x")
        my_y = lax.axis_index("y")
        x_nbr = (1 - my_x, my_y)
        y_nbr = (my_x, 1 - my_y)

        in_copy = pltpu.make_async_copy(x_hbm, xv, in_sem)
        in_copy.start()

        barrier_sem = pltpu.get_barrier_semaphore()
        for nbr in (x_nbr, y_nbr):
            pl.semaphore_signal(
                barrier_sem, inc=1,
                device_id=nbr, device_id_type=pl.DeviceIdType.MESH,
            )
        pl.semaphore_wait(barrier_sem, 2)
        in_copy.wait()

        def chunk(ref, c):
            return ref.at[pl.ds(c * q, q), :]

        def mk_rdma(phase, c, nbr):
            src = chunk(xv if phase == 0 else acc, c)
            sem = N_CHUNKS * phase + c
            return pltpu.make_async_remote_copy(
                src_ref=src,
                dst_ref=recv_buf.at[phase, c],
                send_sem=send_sems.at[sem],
                recv_sem=recv_sems.at[sem],
                device_id=nbr,
                device_id_type=pl.DeviceIdType.MESH,
            )

        nbr0 = [x_nbr] * half + [y_nbr] * half
        nbr1 = [y_nbr] * half + [x_nbr] * half
        order = [c for pair in zip(range(half), range(half, N_CHUNKS))
                 for c in pair]

        p0 = {}
        for c in order:
            p0[c] = mk_rdma(0, c, nbr0[c])
            p0[c].start()
        p1 = {}
        for c in order:
            p0[c].wait()
            chunk(acc, c)[...] = chunk(xv, c)[...] + recv_buf[0, c]
            p1[c] = mk_rdma(1, c, nbr1[c])
            p1[c].start()
        stores = {}
        for c in order:
            p1[c].wait()
            chunk(acc, c)[...] += recv_buf[1, c]
            stores[c] = pltpu.make_async_copy(
                chunk(acc, c), chunk(out_hbm, c), out_sems.at[c]
            )
            stores[c].start()
        for c in order:
            stores[c].wait()

    return pl.pallas_call(
        body,
        out_shape=jax.ShapeDtypeStruct((m, n), jnp.float32),
        in_specs=[pl.BlockSpec(memory_space=pltpu.MemorySpace.HBM)],
        out_specs=pl.BlockSpec(memory_space=pltpu.MemorySpace.HBM),
        scratch_shapes=[
            pltpu.VMEM((m, n), jnp.float32),
            pltpu.VMEM((m, n), jnp.float32),
            pltpu.VMEM((2, N_CHUNKS, q, n), jnp.float32),
            pltpu.SemaphoreType.DMA((2 * N_CHUNKS,)),
            pltpu.SemaphoreType.DMA((2 * N_CHUNKS,)),
            pltpu.SemaphoreType.DMA,
            pltpu.SemaphoreType.DMA((N_CHUNKS,)),
        ],
        compiler_params=pltpu.CompilerParams(collective_id=0),
    )(x)


# device time: 17811 ns/iter; 1.0731x vs baseline; 1.0727x over previous
import jax
import jax.numpy as jnp
from jax import lax
from jax.experimental import pallas as pl
from jax.experimental.pallas import tpu as pltpu

N_CHUNKS = 8


def kernel(x):
    m, n = x.shape
    q = m // N_CHUNKS
    half = N_CHUNKS // 2

    def body(x_hbm, out_hbm, xv, acc, recv_buf, send_sems, recv_sems,
             in_sem, out_sems):
        my_x = lax.axis_index("x")
        my_y = lax.axis_index("y")
        x_nbr = (1 - my_x, my_y)
        y_nbr = (my_x, 1 - my_y)

        in_copy = pltpu.make_async_copy(x_hbm, xv, in_sem)
        in_copy.start()

        barrier_sem = pltpu.get_barrier_semaphore()
        for nbr in (x_nbr, y_nbr):
            pl.semaphore_signal(
                barrier_sem, inc=1,
                device_id=nbr, device_id_type=pl.DeviceIdType.MESH,
            )
        pl.semaphore_wait(barrier_sem, 2)
        in_copy.wait()

        def chunk(ref, c):
            return ref.at[pl.ds(c * q, q), :]

        def mk_rdma(phase, c, nbr):
            src = chunk(xv if phase == 0 else acc, c)
            sem = N_CHUNKS * phase + c
            return pltpu.make_async_remote_copy(
                src_ref=src,
                dst_ref=recv_buf.at[phase, c],
                send_sem=send_sems.at[sem],
                recv_sem=recv_sems.at[sem],
                device_id=nbr,
                device_id_type=pl.DeviceIdType.MESH,
            )

        nbr0 = [x_nbr] * half + [y_nbr] * half
        nbr1 = [y_nbr] * half + [x_nbr] * half
        order = [c for pair in zip(range(half), range(half, N_CHUNKS))
                 for c in pair]

        p0 = {}
        for c in order:
            p0[c] = mk_rdma(0, c, nbr0[c])
            p0[c].start()
        p1 = {}
        for c in order:
            p0[c].wait()
            chunk(acc, c)[...] = chunk(xv, c)[...] + recv_buf[0, c]
            p1[c] = mk_rdma(1, c, nbr1[c])
            p1[c].start()
        stores = {}
        for c in order:
            p1[c].wait()
            chunk(acc, c)[...] += recv_buf[1, c]
            stores[c] = pltpu.make_async_copy(
                chunk(acc, c), chunk(out_hbm, c), out_sems.at[c]
            )
            stores[c].start()
        for c in order:
            stores[c].wait()

    return pl.pallas_call(
        body,
        out_shape=jax.ShapeDtypeStruct((m, n), jnp.float32),
        in_specs=[pl.BlockSpec(memory_space=pltpu.MemorySpace.HBM)],
        out_specs=pl.BlockSpec(memory_space=pltpu.MemorySpace.HBM),
        scratch_shapes=[
            pltpu.VMEM((m, n), jnp.float32),
            pltpu.VMEM((m, n), jnp.float32),
            pltpu.VMEM((2, N_CHUNKS, q, n), jnp.float32),
            pltpu.SemaphoreType.DMA((2 * N_CHUNKS,)),
            pltpu.SemaphoreType.DMA((2 * N_CHUNKS,)),
            pltpu.SemaphoreType.DMA,
            pltpu.SemaphoreType.DMA((N_CHUNKS,)),
        ],
        compiler_params=pltpu.CompilerParams(collective_id=0),
    )(pltpu.with_memory_space_constraint(x, pltpu.MemorySpace.HBM))
